# baseline (speedup 1.0000x reference)
import functools

import jax
import jax.numpy as jnp
from jax import lax
from jax.experimental import pallas as pl
from jax.experimental.pallas import tpu as pltpu
from jax.experimental.pallas import tpu_sc as plsc

N = 10000
E = 320000
D = 128
DH = D // 2
EPS = 1e-5

NC = 2
NS = 16
CH = 80
NCHD = E // (NC * NS * CH)
CHA = 128
NCHA = 158
EPSP = NCHA * CHA
TRASH = N
CHKR = 200
NCHK = N // CHKR


def _zero_vmem(ref, rows, cols):
    zv = jnp.zeros((16,), jnp.float32)
    vpr = cols // 16

    def body(i, _):
        ref[i // vpr, pl.ds((i % vpr) * 16, 16)] = zv
        return 0

    lax.fori_loop(0, rows * vpr, body, 0)


def _my_chunks(s, fn):
    def body(k, _):
        fn((s + NS * k) * CHKR)
        return 0

    lax.fori_loop(0, NCHK // NS, body, 0)

    @pl.when(s < NCHK % NS)
    def _():
        fn((NS * (NCHK // NS) + s) * CHKR)


def _sc_mesh():
    return plsc.VectorSubcoreMesh(core_axis_name="c", subcore_axis_name="s",
                                  num_cores=NC, num_subcores=NS)


def _agg_body(h_hbm, src_hbm, dst_hbm, out_hbm, idx_s, idx_d, rows0, rows1,
              rows2, rows3, zbuf, agg_s, semg0, semg1, semg2, semg3, sems0,
              sems1, sems2, sems3):
    c = lax.axis_index("c")
    s = lax.axis_index("s")
    rows = (rows0, rows1, rows2, rows3)
    semg = (semg0, semg1, semg2, semg3)
    sems = (sems0, sems1, sems2, sems3)

    def start_gather(j, b):
        pltpu.async_copy(h_hbm.at[c].at[idx_s.at[j]], rows[b], semg[b])

    def wait_gather(b):
        pltpu.make_async_copy(h_hbm.at[c].at[idx_s.at[0]], rows[b],
                              semg[b]).wait()

    def start_scatter(j, b):
        pltpu.async_copy(rows[b], agg_s.at[idx_d.at[j]], sems[b], add=True)

    def wait_scatter(b):
        pltpu.make_async_copy(rows[b], agg_s.at[idx_d.at[0]], sems[b]).wait()

    _zero_vmem(zbuf, CHKR, DH)
    _my_chunks(s, lambda r0: pltpu.sync_copy(zbuf, agg_s.at[pl.ds(r0, CHKR)]))

    pltpu.sync_copy(src_hbm.at[s], idx_s)
    pltpu.sync_copy(dst_hbm.at[s], idx_d)
    plsc.subcore_barrier()

    start_gather(0, 0)
    start_gather(1, 1)
    for j in (0, 1):
        wait_gather(j)
        start_scatter(j, j)
        start_gather(j + 2, j + 2)

    def grp(g, _):
        for k in range(4):
            j = 2 + 4 * g + k
            b = (2 + k) % 4
            wait_gather(b)
            start_scatter(j, b)
            b2 = k
            wait_scatter(b2)
            start_gather(j + 2, b2)
        return 0

    lax.fori_loop(0, (NCHA - 6) // 4, grp, 0)
    for j in range(NCHA - 4, NCHA):
        b = j % 4
        wait_gather(b)
        start_scatter(j, b)
        if j + 2 < NCHA:
            b2 = (j + 2) % 4
            wait_scatter(b2)
            start_gather(j + 2, b2)
    for b in range(4):
        wait_scatter(b)
    plsc.subcore_barrier()

    _my_chunks(s, lambda r0: pltpu.sync_copy(
        agg_s.at[pl.ds(r0, CHKR)], out_hbm.at[c, pl.ds(r0, CHKR)]))


@functools.cache
def _agg_call():
    return pl.kernel(
        _agg_body,
        out_type=jax.ShapeDtypeStruct((NC, N, DH), jnp.float32),
        mesh=_sc_mesh(),
        scratch_types=[
            pltpu.VMEM((NCHA, CHA), jnp.int32),
            pltpu.VMEM((NCHA, CHA), jnp.int32),
            pltpu.VMEM((CHA, DH), jnp.float32),
            pltpu.VMEM((CHA, DH), jnp.float32),
            pltpu.VMEM((CHA, DH), jnp.float32),
            pltpu.VMEM((CHA, DH), jnp.float32),
            pltpu.VMEM((CHKR, DH), jnp.float32),
            pltpu.VMEM_SHARED((N + 8, DH), jnp.float32),
        ] + [pltpu.SemaphoreType.DMA] * 8,
        compiler_params=pltpu.CompilerParams(use_tc_tiling_on_sc=False),
    )


def _deg_body(dst_hbm, out_hbm, idx_d, ones, zbuf, deg_s, sem):
    del sem
    c = lax.axis_index("c")
    s = lax.axis_index("s")

    _zero_vmem(zbuf, CHKR, 16)
    _my_chunks(s, lambda r0: pltpu.sync_copy(zbuf, deg_s.at[pl.ds(r0, CHKR)]))

    ov = jnp.ones((16,), jnp.float32)

    def fill(r, _):
        ones[r, pl.ds(0, 16)] = ov
        return 0

    lax.fori_loop(0, CH, fill, 0)

    pltpu.sync_copy(dst_hbm.at[s, c], idx_d)
    plsc.subcore_barrier()

    def chunk(j, _):
        pltpu.sync_copy(ones, deg_s.at[idx_d.at[j]], add=True)
        return 0

    lax.fori_loop(0, NCHD, chunk, 0)
    plsc.subcore_barrier()

    _my_chunks(s, lambda r0: pltpu.sync_copy(
        deg_s.at[pl.ds(r0, CHKR)], out_hbm.at[c, pl.ds(r0, CHKR)]))


@functools.cache
def _deg_call():
    return pl.kernel(
        _deg_body,
        out_type=jax.ShapeDtypeStruct((NC, N, 16), jnp.float32),
        mesh=_sc_mesh(),
        scratch_types=[
            pltpu.VMEM((NCHD, CH), jnp.int32),
            pltpu.VMEM((CH, 16), jnp.float32),
            pltpu.VMEM((CHKR, 16), jnp.float32),
            pltpu.VMEM_SHARED((N, 16), jnp.float32),
            pltpu.SemaphoreType.DMA,
        ],
        compiler_params=pltpu.CompilerParams(use_tc_tiling_on_sc=False),
    )


def _matT(a, w):
    return lax.dot_general(a, w, (((1,), (1,)), ((), ())),
                           preferred_element_type=jnp.float32)


def _bn_relu(z, g, be):
    m = jnp.mean(z, axis=0, keepdims=True)
    zc = z - m
    v = jnp.mean(zc * zc, axis=0, keepdims=True)
    return jnp.maximum(g * zc * jax.lax.rsqrt(v + EPS) + be, 0.0)


def _split_store(o_ref, z):
    o_ref[0] = z[:, :DH]
    o_ref[1] = z[:, DH:]


def _proj_body(x_ref, w_ref, b_ref, g_ref, be_ref, o_ref):
    z = _matT(x_ref[...], w_ref[...]) + b_ref[...]
    _split_store(o_ref, _bn_relu(z, g_ref[...], be_ref[...]))


_H2 = jax.ShapeDtypeStruct((NC, N, DH), jnp.float32)


def _proj_call():
    return pl.pallas_call(_proj_body, out_shape=_H2)


def _layer_body(p_ref, degp_ref, h_ref, wl_ref, bl_ref, wr_ref, g_ref,
                be_ref, o_ref, *, bn):
    deg = degp_ref[0, :, 0:1] + degp_ref[1, :, 0:1]
    inv = 1.0 / jnp.maximum(deg, 1.0)
    agg = jnp.concatenate([p_ref[0], p_ref[1]], axis=1) * inv
    h = jnp.concatenate([h_ref[0], h_ref[1]], axis=1)
    z = _matT(agg, wl_ref[...]) + bl_ref[...] + _matT(h, wr_ref[...])
    if bn:
        _split_store(o_ref, _bn_relu(z, g_ref[...], be_ref[...]))
    else:
        o_ref[...] = z


def _layer_call(bn):
    return pl.pallas_call(
        functools.partial(_layer_body, bn=bn),
        out_shape=_H2 if bn else jax.ShapeDtypeStruct((N, D), jnp.float32),
    )


def kernel(x, W_in, b_in, g_in, be_in, Wl0, bl0, Wr0, g0, be0, Wl1, bl1,
           Wr1, g1, be1, Wl2, bl2, Wr2, edge_index):
    pad = EPSP - E // NS
    src_a = jnp.pad(edge_index[0].reshape(NS, E // NS),
                    ((0, 0), (0, pad))).reshape(NS, NCHA, CHA)
    dst_a = jnp.pad(edge_index[1].reshape(NS, E // NS), ((0, 0), (0, pad)),
                    constant_values=TRASH).reshape(NS, NCHA, CHA)
    dst_d = edge_index[1].reshape(NS, NC, NCHD, CH)

    degp = _deg_call()(dst_d)
    h2 = _proj_call()(x, W_in, b_in.reshape(1, D), g_in.reshape(1, D),
                      be_in.reshape(1, D))

    zd = jnp.zeros((1, D), jnp.float32)
    for Wl, bl, Wr, g, be, last in (
            (Wl0, bl0, Wr0, g0, be0, False),
            (Wl1, bl1, Wr1, g1, be1, False),
            (Wl2, bl2, Wr2, None, None, True),
    ):
        p = _agg_call()(h2, src_a, dst_a)
        gg = zd if last else g.reshape(1, D)
        bb = zd if last else be.reshape(1, D)
        h2 = _layer_call(not last)(p, degp, h2, Wl, bl.reshape(1, D), Wr,
                                   gg, bb)
    return h2


# speedup vs baseline: 1.2634x; 1.2634x over previous
import functools

import jax
import jax.numpy as jnp
from jax import lax
from jax.experimental import pallas as pl
from jax.experimental.pallas import tpu as pltpu
from jax.experimental.pallas import tpu_sc as plsc

N = 10000
E = 320000
D = 128
DH = D // 2
EPS = 1e-5

NC = 2
NS = 16
CH = 80
NCHD = E // (NC * NS * CH)
CHA = 80
NCHA = 250
EPSP = NCHA * CHA
TRASH = N
CHKR = 200
NCHK = N // CHKR


def _zero_vmem(ref, rows, cols):
    zv = jnp.zeros((16,), jnp.float32)
    vpr = cols // 16

    def body(i, _):
        ref[i // vpr, pl.ds((i % vpr) * 16, 16)] = zv
        return 0

    lax.fori_loop(0, rows * vpr, body, 0)


def _my_chunks(s, fn):
    def body(k, _):
        fn((s + NS * k) * CHKR)
        return 0

    lax.fori_loop(0, NCHK // NS, body, 0)

    @pl.when(s < NCHK % NS)
    def _():
        fn((NS * (NCHK // NS) + s) * CHKR)


def _sc_mesh():
    return plsc.VectorSubcoreMesh(core_axis_name="c", subcore_axis_name="s",
                                  num_cores=NC, num_subcores=NS)


def _agg_body(h_hbm, src_hbm, dst_hbm, out_hbm, idx_s, idx_d, rows0, rows1,
              rows2, rows3, zbuf, agg_s, semg0, semg1, semg2, semg3, sems0,
              sems1, sems2, sems3):
    c = lax.axis_index("c")
    s = lax.axis_index("s")
    rows = (rows0, rows1, rows2, rows3)
    semg = (semg0, semg1, semg2, semg3)
    sems = (sems0, sems1, sems2, sems3)

    def start_gather(j, b):
        pltpu.async_copy(h_hbm.at[c].at[idx_s.at[j]], rows[b], semg[b])

    def wait_gather(b):
        pltpu.make_async_copy(h_hbm.at[c].at[idx_s.at[0]], rows[b],
                              semg[b]).wait()

    def start_scatter(j, b):
        pltpu.async_copy(rows[b], agg_s.at[idx_d.at[j]], sems[b], add=True)

    def wait_scatter(b):
        pltpu.make_async_copy(rows[b], agg_s.at[idx_d.at[0]], sems[b]).wait()

    _zero_vmem(zbuf, CHKR, DH)
    _my_chunks(s, lambda r0: pltpu.sync_copy(zbuf, agg_s.at[pl.ds(r0, CHKR)]))

    pltpu.sync_copy(src_hbm.at[s], idx_s)
    pltpu.sync_copy(dst_hbm.at[s], idx_d)
    plsc.subcore_barrier()

    start_gather(0, 0)
    start_gather(1, 1)
    for j in (0, 1):
        wait_gather(j)
        start_scatter(j, j)
        start_gather(j + 2, j + 2)

    def grp(g, _):
        for k in range(4):
            j = 2 + 4 * g + k
            b = (2 + k) % 4
            wait_gather(b)
            start_scatter(j, b)
            b2 = k
            wait_scatter(b2)
            start_gather(j + 2, b2)
        return 0

    lax.fori_loop(0, (NCHA - 6) // 4, grp, 0)
    for j in range(NCHA - 4, NCHA):
        b = j % 4
        wait_gather(b)
        start_scatter(j, b)
        if j + 2 < NCHA:
            b2 = (j + 2) % 4
            wait_scatter(b2)
            start_gather(j + 2, b2)
    for b in range(4):
        wait_scatter(b)
    plsc.subcore_barrier()

    _my_chunks(s, lambda r0: pltpu.sync_copy(
        agg_s.at[pl.ds(r0, CHKR)], out_hbm.at[c, pl.ds(r0, CHKR)]))


@functools.cache
def _agg_call():
    return pl.kernel(
        _agg_body,
        out_type=jax.ShapeDtypeStruct((NC, N, DH), jnp.float32),
        mesh=_sc_mesh(),
        scratch_types=[
            pltpu.VMEM((NCHA, CHA), jnp.int32),
            pltpu.VMEM((NCHA, CHA), jnp.int32),
            pltpu.VMEM((CHA, DH), jnp.float32),
            pltpu.VMEM((CHA, DH), jnp.float32),
            pltpu.VMEM((CHA, DH), jnp.float32),
            pltpu.VMEM((CHA, DH), jnp.float32),
            pltpu.VMEM((CHKR, DH), jnp.float32),
            pltpu.VMEM_SHARED((N + 8, DH), jnp.float32),
        ] + [pltpu.SemaphoreType.DMA] * 8,
        compiler_params=pltpu.CompilerParams(use_tc_tiling_on_sc=False),
    )


def _deg_body(dst_hbm, out_hbm, idx_d, ones, zbuf, deg_s, sem):
    del sem
    c = lax.axis_index("c")
    s = lax.axis_index("s")

    _zero_vmem(zbuf, CHKR, 16)
    _my_chunks(s, lambda r0: pltpu.sync_copy(zbuf, deg_s.at[pl.ds(r0, CHKR)]))

    ov = jnp.ones((16,), jnp.float32)

    def fill(r, _):
        ones[r, pl.ds(0, 16)] = ov
        return 0

    lax.fori_loop(0, CH, fill, 0)

    pltpu.sync_copy(dst_hbm.at[s, c], idx_d)
    plsc.subcore_barrier()

    def chunk(j, _):
        pltpu.sync_copy(ones, deg_s.at[idx_d.at[j]], add=True)
        return 0

    lax.fori_loop(0, NCHD, chunk, 0)
    plsc.subcore_barrier()

    _my_chunks(s, lambda r0: pltpu.sync_copy(
        deg_s.at[pl.ds(r0, CHKR)], out_hbm.at[c, pl.ds(r0, CHKR)]))


@functools.cache
def _deg_call():
    return pl.kernel(
        _deg_body,
        out_type=jax.ShapeDtypeStruct((NC, N, 16), jnp.float32),
        mesh=_sc_mesh(),
        scratch_types=[
            pltpu.VMEM((NCHD, CH), jnp.int32),
            pltpu.VMEM((CH, 16), jnp.float32),
            pltpu.VMEM((CHKR, 16), jnp.float32),
            pltpu.VMEM_SHARED((N, 16), jnp.float32),
            pltpu.SemaphoreType.DMA,
        ],
        compiler_params=pltpu.CompilerParams(use_tc_tiling_on_sc=False),
    )


def _matT(a, w):
    return lax.dot_general(a, w, (((1,), (1,)), ((), ())),
                           preferred_element_type=jnp.float32)


def _bn_relu(z, g, be):
    m = jnp.mean(z, axis=0, keepdims=True)
    zc = z - m
    v = jnp.mean(zc * zc, axis=0, keepdims=True)
    return jnp.maximum(g * zc * jax.lax.rsqrt(v + EPS) + be, 0.0)


def _split_store(o_ref, z):
    o_ref[0] = z[:, :DH]
    o_ref[1] = z[:, DH:]


def _proj_body(x_ref, w_ref, b_ref, g_ref, be_ref, o_ref):
    z = _matT(x_ref[...], w_ref[...]) + b_ref[...]
    _split_store(o_ref, _bn_relu(z, g_ref[...], be_ref[...]))


_H2 = jax.ShapeDtypeStruct((NC, N, DH), jnp.float32)


def _proj_call():
    return pl.pallas_call(_proj_body, out_shape=_H2)


def _layer_body(p_ref, degp_ref, h_ref, wl_ref, bl_ref, wr_ref, g_ref,
                be_ref, o_ref, *, bn):
    deg = degp_ref[0, :, 0:1] + degp_ref[1, :, 0:1]
    inv = 1.0 / jnp.maximum(deg, 1.0)
    agg = jnp.concatenate([p_ref[0], p_ref[1]], axis=1) * inv
    h = jnp.concatenate([h_ref[0], h_ref[1]], axis=1)
    z = _matT(agg, wl_ref[...]) + bl_ref[...] + _matT(h, wr_ref[...])
    if bn:
        _split_store(o_ref, _bn_relu(z, g_ref[...], be_ref[...]))
    else:
        o_ref[...] = z


def _layer_call(bn):
    return pl.pallas_call(
        functools.partial(_layer_body, bn=bn),
        out_shape=_H2 if bn else jax.ShapeDtypeStruct((N, D), jnp.float32),
    )


def kernel(x, W_in, b_in, g_in, be_in, Wl0, bl0, Wr0, g0, be0, Wl1, bl1,
           Wr1, g1, be1, Wl2, bl2, Wr2, edge_index):
    pad = EPSP - E // NS
    src_a = jnp.pad(edge_index[0].reshape(NS, E // NS),
                    ((0, 0), (0, pad))).reshape(NS, NCHA, CHA)
    dst_a = jnp.pad(edge_index[1].reshape(NS, E // NS), ((0, 0), (0, pad)),
                    constant_values=TRASH).reshape(NS, NCHA, CHA)
    dst_d = edge_index[1].reshape(NS, NC, NCHD, CH)

    degp = _deg_call()(dst_d)
    h2 = _proj_call()(x, W_in, b_in.reshape(1, D), g_in.reshape(1, D),
                      be_in.reshape(1, D))

    zd = jnp.zeros((1, D), jnp.float32)
    for Wl, bl, Wr, g, be, last in (
            (Wl0, bl0, Wr0, g0, be0, False),
            (Wl1, bl1, Wr1, g1, be1, False),
            (Wl2, bl2, Wr2, None, None, True),
    ):
        p = _agg_call()(h2, src_a, dst_a)
        gg = zd if last else g.reshape(1, D)
        bb = zd if last else be.reshape(1, D)
        h2 = _layer_call(not last)(p, degp, h2, Wl, bl.reshape(1, D), Wr,
                                   gg, bb)
    return h2


# speedup vs baseline: 1.3281x; 1.0512x over previous
import functools

import jax
import jax.numpy as jnp
from jax import lax
from jax.experimental import pallas as pl
from jax.experimental.pallas import tpu as pltpu
from jax.experimental.pallas import tpu_sc as plsc

N = 10000
E = 320000
D = 128
DH = D // 2
EPS = 1e-5

NC = 2
NS = 16
CH = 80
NCHD = E // (NC * NS * CH)
CHA = 80
NCHA = 250
CHKR = 200
NCHK = N // CHKR


def _zero_vmem(ref, rows, cols):
    zv = jnp.zeros((16,), jnp.float32)
    vpr = cols // 16

    def body(i, _):
        ref[i // vpr, pl.ds((i % vpr) * 16, 16)] = zv
        return 0

    lax.fori_loop(0, rows * vpr, body, 0)


def _my_chunks(s, fn):
    def body(k, _):
        fn((s + NS * k) * CHKR)
        return 0

    lax.fori_loop(0, NCHK // NS, body, 0)

    @pl.when(s < NCHK % NS)
    def _():
        fn((NS * (NCHK // NS) + s) * CHKR)


def _sc_mesh():
    return plsc.VectorSubcoreMesh(core_axis_name="c", subcore_axis_name="s",
                                  num_cores=NC, num_subcores=NS)


def _agg_body(h_hbm, e_hbm, out_hbm, idx_s, idx_d, rows0, rows1,
              rows2, rows3, zbuf, agg_s, semg0, semg1, semg2, semg3, sems0,
              sems1, sems2, sems3):
    c = lax.axis_index("c")
    s = lax.axis_index("s")
    rows = (rows0, rows1, rows2, rows3)
    semg = (semg0, semg1, semg2, semg3)
    sems = (sems0, sems1, sems2, sems3)

    def start_gather(j, b):
        pltpu.async_copy(h_hbm.at[c].at[idx_s.at[j]], rows[b], semg[b])

    def wait_gather(b):
        pltpu.make_async_copy(h_hbm.at[c].at[idx_s.at[0]], rows[b],
                              semg[b]).wait()

    def start_scatter(j, b):
        pltpu.async_copy(rows[b], agg_s.at[idx_d.at[j]], sems[b], add=True)

    def wait_scatter(b):
        pltpu.make_async_copy(rows[b], agg_s.at[idx_d.at[0]], sems[b]).wait()

    _zero_vmem(zbuf, CHKR, DH)
    _my_chunks(s, lambda r0: pltpu.sync_copy(zbuf, agg_s.at[pl.ds(r0, CHKR)]))

    pltpu.sync_copy(e_hbm.at[0, s], idx_s)
    pltpu.sync_copy(e_hbm.at[1, s], idx_d)
    plsc.subcore_barrier()

    start_gather(0, 0)
    start_gather(1, 1)
    for j in (0, 1):
        wait_gather(j)
        start_scatter(j, j)
        start_gather(j + 2, j + 2)

    def grp(g, _):
        for k in range(4):
            j = 2 + 4 * g + k
            b = (2 + k) % 4
            wait_gather(b)
            start_scatter(j, b)
            b2 = k
            wait_scatter(b2)
            start_gather(j + 2, b2)
        return 0

    lax.fori_loop(0, (NCHA - 6) // 4, grp, 0)
    for j in range(NCHA - 4, NCHA):
        b = j % 4
        wait_gather(b)
        start_scatter(j, b)
        if j + 2 < NCHA:
            b2 = (j + 2) % 4
            wait_scatter(b2)
            start_gather(j + 2, b2)
    for b in range(4):
        wait_scatter(b)
    plsc.subcore_barrier()

    _my_chunks(s, lambda r0: pltpu.sync_copy(
        agg_s.at[pl.ds(r0, CHKR)], out_hbm.at[c, pl.ds(r0, CHKR)]))


@functools.cache
def _agg_call():
    return pl.kernel(
        _agg_body,
        out_type=jax.ShapeDtypeStruct((NC, N, DH), jnp.float32),
        mesh=_sc_mesh(),
        scratch_types=[
            pltpu.VMEM((NCHA, CHA), jnp.int32),
            pltpu.VMEM((NCHA, CHA), jnp.int32),
            pltpu.VMEM((CHA, DH), jnp.float32),
            pltpu.VMEM((CHA, DH), jnp.float32),
            pltpu.VMEM((CHA, DH), jnp.float32),
            pltpu.VMEM((CHA, DH), jnp.float32),
            pltpu.VMEM((CHKR, DH), jnp.float32),
            pltpu.VMEM_SHARED((N, DH), jnp.float32),
        ] + [pltpu.SemaphoreType.DMA] * 8,
        compiler_params=pltpu.CompilerParams(use_tc_tiling_on_sc=False),
    )


def _deg_body(e_hbm, out_hbm, idx_d, ones, zbuf, deg_s, sem):
    del sem
    c = lax.axis_index("c")
    s = lax.axis_index("s")

    _zero_vmem(zbuf, CHKR, 16)
    _my_chunks(s, lambda r0: pltpu.sync_copy(zbuf, deg_s.at[pl.ds(r0, CHKR)]))

    ov = jnp.ones((16,), jnp.float32)

    def fill(r, _):
        ones[r, pl.ds(0, 16)] = ov
        return 0

    lax.fori_loop(0, CH, fill, 0)

    pltpu.sync_copy(e_hbm.at[1, s], idx_d)
    plsc.subcore_barrier()

    def chunk(j, _):
        pltpu.sync_copy(ones, deg_s.at[idx_d.at[j]], add=True)
        return 0

    lax.fori_loop(c * NCHD, (c + 1) * NCHD, chunk, 0)
    plsc.subcore_barrier()

    _my_chunks(s, lambda r0: pltpu.sync_copy(
        deg_s.at[pl.ds(r0, CHKR)], out_hbm.at[c, pl.ds(r0, CHKR)]))


@functools.cache
def _deg_call():
    return pl.kernel(
        _deg_body,
        out_type=jax.ShapeDtypeStruct((NC, N, 16), jnp.float32),
        mesh=_sc_mesh(),
        scratch_types=[
            pltpu.VMEM((NCHA, CHA), jnp.int32),
            pltpu.VMEM((CHA, 16), jnp.float32),
            pltpu.VMEM((CHKR, 16), jnp.float32),
            pltpu.VMEM_SHARED((N, 16), jnp.float32),
            pltpu.SemaphoreType.DMA,
        ],
        compiler_params=pltpu.CompilerParams(use_tc_tiling_on_sc=False),
    )


def _matT(a, w):
    return lax.dot_general(a, w, (((1,), (1,)), ((), ())),
                           preferred_element_type=jnp.float32)


def _bn_relu(z, g, be):
    m = jnp.mean(z, axis=0, keepdims=True)
    zc = z - m
    v = jnp.mean(zc * zc, axis=0, keepdims=True)
    return jnp.maximum(g * zc * jax.lax.rsqrt(v + EPS) + be, 0.0)


def _split_store(o_ref, z):
    o_ref[0] = z[:, :DH]
    o_ref[1] = z[:, DH:]


def _proj_body(x_ref, w_ref, b_ref, g_ref, be_ref, o_ref):
    z = _matT(x_ref[...], w_ref[...]) + b_ref[...]
    _split_store(o_ref, _bn_relu(z, g_ref[...], be_ref[...]))


_H2 = jax.ShapeDtypeStruct((NC, N, DH), jnp.float32)


def _proj_call():
    return pl.pallas_call(_proj_body, out_shape=_H2)


def _layer_body(p_ref, degp_ref, h_ref, wl_ref, bl_ref, wr_ref, g_ref,
                be_ref, o_ref, *, bn):
    deg = degp_ref[0, :, 0:1] + degp_ref[1, :, 0:1]
    inv = 1.0 / jnp.maximum(deg, 1.0)
    agg = jnp.concatenate([p_ref[0], p_ref[1]], axis=1) * inv
    h = jnp.concatenate([h_ref[0], h_ref[1]], axis=1)
    z = _matT(agg, wl_ref[...]) + bl_ref[...] + _matT(h, wr_ref[...])
    if bn:
        _split_store(o_ref, _bn_relu(z, g_ref[...], be_ref[...]))
    else:
        o_ref[...] = z


def _layer_call(bn):
    return pl.pallas_call(
        functools.partial(_layer_body, bn=bn),
        out_shape=_H2 if bn else jax.ShapeDtypeStruct((N, D), jnp.float32),
    )


def kernel(x, W_in, b_in, g_in, be_in, Wl0, bl0, Wr0, g0, be0, Wl1, bl1,
           Wr1, g1, be1, Wl2, bl2, Wr2, edge_index):
    e4 = edge_index.reshape(2, NS, NCHA, CHA)

    degp = _deg_call()(e4)
    h2 = _proj_call()(x, W_in, b_in.reshape(1, D), g_in.reshape(1, D),
                      be_in.reshape(1, D))

    zd = jnp.zeros((1, D), jnp.float32)
    for Wl, bl, Wr, g, be, last in (
            (Wl0, bl0, Wr0, g0, be0, False),
            (Wl1, bl1, Wr1, g1, be1, False),
            (Wl2, bl2, Wr2, None, None, True),
    ):
        p = _agg_call()(h2, e4)
        gg = zd if last else g.reshape(1, D)
        bb = zd if last else be.reshape(1, D)
        h2 = _layer_call(not last)(p, degp, h2, Wl, bl.reshape(1, D), Wr,
                                   gg, bb)
    return h2


# speedup vs baseline: 1.5607x; 1.1751x over previous
import functools

import jax
import jax.numpy as jnp
from jax import lax
from jax.experimental import pallas as pl
from jax.experimental.pallas import tpu as pltpu
from jax.experimental.pallas import tpu_sc as plsc

N = 10000
E = 320000
D = 128
DH = D // 2
EPS = 1e-5

NC = 2
NS = 16
CH = 80
NCHD = E // (NC * NS * CH)
CHA = 80
NCHA = 250
CHKR = 80
NCHK = N // CHKR


def _zero_vmem(ref, rows, cols):
    zv = jnp.zeros((16,), jnp.float32)
    vpr = cols // 16

    def body(i, _):
        ref[i // vpr, pl.ds((i % vpr) * 16, 16)] = zv
        return 0

    lax.fori_loop(0, rows * vpr, body, 0)


def _my_chunks(s, fn):
    def body(k, _):
        fn((s + NS * k) * CHKR)
        return 0

    lax.fori_loop(0, NCHK // NS, body, 0)

    @pl.when(s < NCHK % NS)
    def _():
        fn((NS * (NCHK // NS) + s) * CHKR)


def _sc_mesh():
    return plsc.VectorSubcoreMesh(core_axis_name="c", subcore_axis_name="s",
                                  num_cores=NC, num_subcores=NS)


NB = 8
LA = NB // 2


def _agg_body(h_hbm, e_hbm, out_hbm, idx_s, idx_d, *rest):
    rows = rest[:NB]
    agg_s = rest[NB]
    semg = rest[NB + 1:2 * NB + 1]
    sems = rest[2 * NB + 1:]
    c = lax.axis_index("c")
    s = lax.axis_index("s")

    def start_gather(j, b):
        pltpu.async_copy(h_hbm.at[c].at[idx_s.at[j]], rows[b], semg[b])

    def wait_gather(b):
        pltpu.make_async_copy(h_hbm.at[c].at[idx_s.at[0]], rows[b],
                              semg[b]).wait()

    def start_scatter(j, b):
        pltpu.async_copy(rows[b], agg_s.at[idx_d.at[j]], sems[b], add=True)

    def wait_scatter(b):
        pltpu.make_async_copy(rows[b], agg_s.at[idx_d.at[0]], sems[b]).wait()

    _zero_vmem(rows[0], CHA, DH)
    _my_chunks(s, lambda r0: pltpu.sync_copy(rows[0],
                                             agg_s.at[pl.ds(r0, CHKR)]))

    pltpu.sync_copy(e_hbm.at[0, s], idx_s)
    pltpu.sync_copy(e_hbm.at[1, s], idx_d)
    plsc.subcore_barrier()

    ngrp = (NCHA - LA - NB) // NB
    ntail = NCHA - LA - ngrp * NB
    for j in range(LA):
        start_gather(j, j)
    for j in range(LA):
        wait_gather(j)
        start_scatter(j, j)
        start_gather(j + LA, j + LA)

    def grp(g, _):
        for k in range(NB):
            j = LA + NB * g + k
            b = (LA + k) % NB
            wait_gather(b)
            start_scatter(j, b)
            b2 = k
            wait_scatter(b2)
            start_gather(j + LA, b2)
        return 0

    lax.fori_loop(0, ngrp, grp, 0)
    for jj in range(ntail):
        j = LA + ngrp * NB + jj
        b = j % NB
        wait_gather(b)
        start_scatter(j, b)
        if j + LA < NCHA:
            b2 = (j + LA) % NB
            wait_scatter(b2)
            start_gather(j + LA, b2)
    for b in range(NB):
        wait_scatter(b)
    plsc.subcore_barrier()

    _my_chunks(s, lambda r0: pltpu.sync_copy(
        agg_s.at[pl.ds(r0, CHKR)], out_hbm.at[c, pl.ds(r0, CHKR)]))


@functools.cache
def _agg_call():
    return pl.kernel(
        _agg_body,
        out_type=jax.ShapeDtypeStruct((NC, N, DH), jnp.float32),
        mesh=_sc_mesh(),
        scratch_types=[
            pltpu.VMEM((NCHA, CHA), jnp.int32),
            pltpu.VMEM((NCHA, CHA), jnp.int32),
        ] + [pltpu.VMEM((CHA, DH), jnp.float32)] * NB
        + [
            pltpu.VMEM_SHARED((N, DH), jnp.float32),
        ] + [pltpu.SemaphoreType.DMA] * (2 * NB),
        compiler_params=pltpu.CompilerParams(use_tc_tiling_on_sc=False),
    )


def _deg_body(e_hbm, out_hbm, idx_d, ones, zbuf, deg_s, sem):
    del sem
    c = lax.axis_index("c")
    s = lax.axis_index("s")

    _zero_vmem(zbuf, CHKR, 16)
    _my_chunks(s, lambda r0: pltpu.sync_copy(zbuf, deg_s.at[pl.ds(r0, CHKR)]))

    ov = jnp.ones((16,), jnp.float32)

    def fill(r, _):
        ones[r, pl.ds(0, 16)] = ov
        return 0

    lax.fori_loop(0, CH, fill, 0)

    pltpu.sync_copy(e_hbm.at[1, s], idx_d)
    plsc.subcore_barrier()

    def chunk(j, _):
        pltpu.sync_copy(ones, deg_s.at[idx_d.at[j]], add=True)
        return 0

    lax.fori_loop(c * NCHD, (c + 1) * NCHD, chunk, 0)
    plsc.subcore_barrier()

    _my_chunks(s, lambda r0: pltpu.sync_copy(
        deg_s.at[pl.ds(r0, CHKR)], out_hbm.at[c, pl.ds(r0, CHKR)]))


@functools.cache
def _deg_call():
    return pl.kernel(
        _deg_body,
        out_type=jax.ShapeDtypeStruct((NC, N, 16), jnp.float32),
        mesh=_sc_mesh(),
        scratch_types=[
            pltpu.VMEM((NCHA, CHA), jnp.int32),
            pltpu.VMEM((CHA, 16), jnp.float32),
            pltpu.VMEM((CHKR, 16), jnp.float32),
            pltpu.VMEM_SHARED((N, 16), jnp.float32),
            pltpu.SemaphoreType.DMA,
        ],
        compiler_params=pltpu.CompilerParams(use_tc_tiling_on_sc=False),
    )


def _matT(a, w):
    return lax.dot_general(a, w, (((1,), (1,)), ((), ())),
                           preferred_element_type=jnp.float32)


def _bn_relu(z, g, be):
    m = jnp.mean(z, axis=0, keepdims=True)
    zc = z - m
    v = jnp.mean(zc * zc, axis=0, keepdims=True)
    return jnp.maximum(g * zc * jax.lax.rsqrt(v + EPS) + be, 0.0)


def _split_store(o_ref, z):
    o_ref[0] = z[:, :DH]
    o_ref[1] = z[:, DH:]


def _proj_body(x_ref, w_ref, b_ref, g_ref, be_ref, o_ref):
    z = _matT(x_ref[...], w_ref[...]) + b_ref[...]
    _split_store(o_ref, _bn_relu(z, g_ref[...], be_ref[...]))


_H2 = jax.ShapeDtypeStruct((NC, N, DH), jnp.float32)


def _proj_call():
    return pl.pallas_call(_proj_body, out_shape=_H2)


def _layer_body(p_ref, degp_ref, h_ref, wl_ref, bl_ref, wr_ref, g_ref,
                be_ref, o_ref, *, bn):
    deg = degp_ref[0, :, 0:1] + degp_ref[1, :, 0:1]
    inv = 1.0 / jnp.maximum(deg, 1.0)
    agg = jnp.concatenate([p_ref[0], p_ref[1]], axis=1) * inv
    h = jnp.concatenate([h_ref[0], h_ref[1]], axis=1)
    z = _matT(agg, wl_ref[...]) + bl_ref[...] + _matT(h, wr_ref[...])
    if bn:
        _split_store(o_ref, _bn_relu(z, g_ref[...], be_ref[...]))
    else:
        o_ref[...] = z


def _layer_call(bn):
    return pl.pallas_call(
        functools.partial(_layer_body, bn=bn),
        out_shape=_H2 if bn else jax.ShapeDtypeStruct((N, D), jnp.float32),
    )


def kernel(x, W_in, b_in, g_in, be_in, Wl0, bl0, Wr0, g0, be0, Wl1, bl1,
           Wr1, g1, be1, Wl2, bl2, Wr2, edge_index):
    e4 = edge_index.reshape(2, NS, NCHA, CHA)

    degp = _deg_call()(e4)
    h2 = _proj_call()(x, W_in, b_in.reshape(1, D), g_in.reshape(1, D),
                      be_in.reshape(1, D))

    zd = jnp.zeros((1, D), jnp.float32)
    for Wl, bl, Wr, g, be, last in (
            (Wl0, bl0, Wr0, g0, be0, False),
            (Wl1, bl1, Wr1, g1, be1, False),
            (Wl2, bl2, Wr2, None, None, True),
    ):
        p = _agg_call()(h2, e4)
        gg = zd if last else g.reshape(1, D)
        bb = zd if last else be.reshape(1, D)
        h2 = _layer_call(not last)(p, degp, h2, Wl, bl.reshape(1, D), Wr,
                                   gg, bb)
    return h2
